# Initial kernel scaffold; baseline (speedup 1.0000x reference)
import jax
import jax.numpy as jnp
from jax.experimental import pallas as pl


def kernel(obs, reward, adjacency, hidden_state, post_neuron_state, lr, Wn_in, bn_in, Wn_hid, bn_hid, Wn_out, bn_out, Wih_in, Whh_in, bih_in, bhh_in, Wih_hid, Whh_hid, bih_hid, bhh_hid, Wih_out, Whh_out, bih_out, bhh_out):
    raise NotImplementedError("write your pallas kernel here")


# speedup vs baseline: 3.4345x; 3.4345x over previous
import jax
import jax.numpy as jnp
from jax.experimental import pallas as pl
from jax.experimental.pallas import tpu as pltpu

I_SZ, H_SZ, O_SZ = 32, 192, 32
N = I_SZ + H_SZ + O_SZ
NP = N // 2
D = 64
P = 2 * D
GP = 6 * D
TICKS = 2
RC = 8


def _sfnn_body(rew_ref, lr_ref, in_sig2_ref, post20_ref, adjte_ref, adjto_ref,
               ee_ref, eo_ref, hid_hbm,
               wsum_in2_ref, bn_in2_ref, wn_hid2_ref, bn_hid2_ref,
               wn_out2_ref, bn_out2_ref,
               w1_in, w2_in, wr_in, bi_in, whh_in, bh_in,
               w1_hid, w2_hid, wr_hid, bi_hid, whh_hid, bh_hid,
               w1_out, w2_out, wr_out, bi_out, whh_out, bh_out,
               out_ref,
               hid_ref, sy_ref, s_ref, post2_ref, dma_sem):
    cp = pltpu.make_async_copy(hid_hbm, hid_ref, dma_sem)
    cp.start()
    post2_ref[...] = post20_ref[...]
    sy_ref[...] = jnp.zeros((N * NP, P), jnp.float32)
    s_ref[...] = jnp.zeros((NP, P), jnp.float32)
    cp.wait()

    rew = rew_ref[0, 0].astype(jnp.bfloat16).astype(jnp.float32)
    lr = lr_ref[0, 0]
    ee = ee_ref[...]
    eo = eo_ref[...]

    def unpack_post():
        p2 = post2_ref[...]
        hi = jax.lax.Precision.HIGHEST
        return (jnp.dot(ee, p2[:, :D], preferred_element_type=jnp.float32,
                        precision=hi)
                + jnp.dot(eo, p2[:, D:], preferred_element_type=jnp.float32,
                          precision=hi))

    def neuron_update(lo2, sz2, x2, wnt2_ref, bn2_ref):
        y = jnp.dot(x2, wnt2_ref[...], preferred_element_type=jnp.float32)
        post2_ref[lo2:lo2 + sz2, :] = jnp.tanh(y + bn2_ref[...])

    def syn_stage(r0, R, w1t2_ref, w2t2_ref, wr2_ref, bi2_ref, whht2_ref, bh2_ref):
        postu = unpack_post()
        pre_all = postu[r0:r0 + R, :]
        A2 = (jnp.dot(pre_all, w1t2_ref[...], preferred_element_type=jnp.float32)
              + bi2_ref[...] + rew * wr2_ref[...])
        B2 = jnp.dot(post2_ref[...], w2t2_ref[...],
                     preferred_element_type=jnp.float32)
        whht2 = whht2_ref[...]
        bh2 = bh2_ref[...]

        for i in range(R // RC):
            rs = r0 + i * RC
            cs = rs * NP
            h = hid_ref[cs:cs + RC * NP, :]
            gh = (jnp.dot(h, whht2, preferred_element_type=jnp.float32)
                  + bh2).reshape(RC, NP, GP)
            a_c = A2[i * RC:(i + 1) * RC, :].reshape(RC, 1, GP)
            gi = a_c + B2[None, :, :]
            h3 = h.reshape(RC, NP, P)
            r = jax.nn.sigmoid(gi[..., :P] + gh[..., :P])
            z = jax.nn.sigmoid(gi[..., P:2 * P] + gh[..., P:2 * P])
            n = jnp.tanh(gi[..., 2 * P:] + r * gh[..., 2 * P:])
            g = (1.0 - z) * n + z * h3
            upd = h3 + g * lr
            hid_ref[cs:cs + RC * NP, :] = upd.reshape(RC * NP, P)
            pre_c = pre_all[i * RC:(i + 1) * RC, :]
            pre2 = jnp.concatenate([pre_c, pre_c], axis=1).reshape(RC, 1, P)
            syn = upd * pre2
            old = sy_ref[cs:cs + RC * NP, :].reshape(RC, NP, P)
            acc = s_ref[...]
            for j in range(RC):
                rr = rs + j
                me = adjte_ref[:, rr:rr + 1]
                mo = adjto_ref[:, rr:rr + 1]
                msyn = jnp.concatenate(
                    [me * syn[j, :, :D], mo * syn[j, :, D:]], axis=1)
                acc = acc + (msyn - old[j])
                sy_ref[cs + j * NP:cs + (j + 1) * NP, :] = msyn
            s_ref[...] = acc

    for _ in range(TICKS):
        post2_ref[0:I_SZ // 2, :] = jnp.tanh(
            in_sig2_ref[...] * wsum_in2_ref[...] + bn_in2_ref[...])
        syn_stage(0, I_SZ, w1_in, w2_in, wr_in, bi_in, whh_in, bh_in)
        neuron_update(I_SZ // 2, H_SZ // 2, s_ref[I_SZ // 2:(I_SZ + H_SZ) // 2, :],
                      wn_hid2_ref, bn_hid2_ref)
        syn_stage(I_SZ, H_SZ, w1_hid, w2_hid, wr_hid, bi_hid, whh_hid, bh_hid)
        neuron_update((N - O_SZ) // 2, O_SZ // 2, s_ref[(N - O_SZ) // 2:, :],
                      wn_out2_ref, bn_out2_ref)
        syn_stage(N - O_SZ, O_SZ, w1_out, w2_out, wr_out, bi_out, whh_out, bh_out)

    postu = unpack_post()
    v = postu[N - O_SZ:, 0:1]
    mx = jnp.max(v)
    iota = jax.lax.broadcasted_iota(jnp.int32, (O_SZ, 1), 0)
    out_ref[0, 0] = jnp.min(jnp.where(v == mx, iota, N))


def kernel(obs, reward, adjacency, hidden_state, post_neuron_state, lr,
           Wn_in, bn_in, Wn_hid, bn_hid, Wn_out, bn_out,
           Wih_in, Whh_in, bih_in, bhh_in,
           Wih_hid, Whh_hid, bih_hid, bhh_hid,
           Wih_out, Whh_out, bih_out, bhh_out):
    f32 = jnp.float32
    in_sig2 = jnp.broadcast_to(obs.reshape(I_SZ, 1), (I_SZ, D)).reshape(I_SZ // 2, P)
    hid2 = hidden_state.reshape(N * NP, P)
    post20 = post_neuron_state.reshape(NP, P)
    adjf = adjacency.astype(f32).T
    adjte = adjf[0::2, :]
    adjto = adjf[1::2, :]
    ridx = jnp.arange(N)[:, None]
    kidx = jnp.arange(NP)[None, :]
    ee = (ridx == 2 * kidx).astype(f32)
    eo = (ridx == 2 * kidx + 1).astype(f32)

    def dupg(v):
        return jnp.concatenate([v[0:D], v[0:D], v[D:2 * D], v[D:2 * D],
                                v[2 * D:], v[2 * D:]]).reshape(1, GP)

    def dupw(W):
        Wt = W.T
        return jnp.concatenate([Wt[:, 0:D], Wt[:, 0:D], Wt[:, D:2 * D],
                                Wt[:, D:2 * D], Wt[:, 2 * D:], Wt[:, 2 * D:]], axis=1)

    Z = jnp.zeros((D, D), f32)

    def blkw(W):
        Wt = W.T
        top = jnp.concatenate([Wt[:, 0:D], Z, Wt[:, D:2 * D], Z, Wt[:, 2 * D:], Z], axis=1)
        bot = jnp.concatenate([Z, Wt[:, 0:D], Z, Wt[:, D:2 * D], Z, Wt[:, 2 * D:]], axis=1)
        return jnp.concatenate([top, bot], axis=0)

    def blkn(Wn, bn):
        Wt = Wn.T
        w2 = jnp.concatenate([jnp.concatenate([Wt, Z], axis=1),
                              jnp.concatenate([Z, Wt], axis=1)], axis=0)
        return w2, jnp.concatenate([bn, bn]).reshape(1, P)

    wsum = Wn_in.sum(axis=1)
    wsum_in2 = jnp.concatenate([wsum, wsum]).reshape(1, P)
    bn_in2 = jnp.concatenate([bn_in, bn_in]).reshape(1, P)
    wn_hid2, bn_hid2 = blkn(Wn_hid, bn_hid)
    wn_out2, bn_out2 = blkn(Wn_out, bn_out)

    def gru_w(Wih, Whh, bih, bhh):
        w1t2 = dupw(Wih[:, :D])
        w2t2 = blkw(Wih[:, D:2 * D])
        wr2 = dupg(Wih[:, 2 * D].astype(jnp.bfloat16).astype(f32))
        bi2 = dupg(bih)
        whht2 = blkw(Whh)
        bh2 = dupg(bhh)
        return w1t2, w2t2, wr2, bi2, whht2, bh2

    args = [reward.reshape(1, 1), jnp.asarray(lr, f32).reshape(1, 1),
            in_sig2, post20, adjte, adjto, ee, eo, hid2,
            wsum_in2, bn_in2, wn_hid2, bn_hid2, wn_out2, bn_out2]
    args += list(gru_w(Wih_in, Whh_in, bih_in, bhh_in))
    args += list(gru_w(Wih_hid, Whh_hid, bih_hid, bhh_hid))
    args += list(gru_w(Wih_out, Whh_out, bih_out, bhh_out))

    smem = pl.BlockSpec(memory_space=pltpu.MemorySpace.SMEM)
    vmem = pl.BlockSpec(memory_space=pltpu.MemorySpace.VMEM)
    hbm = pl.BlockSpec(memory_space=pltpu.MemorySpace.HBM)
    in_specs = [smem, smem] + [vmem] * 6 + [hbm] + [vmem] * 24

    out = pl.pallas_call(
        _sfnn_body,
        out_shape=jax.ShapeDtypeStruct((1, 1), jnp.int32),
        in_specs=in_specs,
        out_specs=pl.BlockSpec(memory_space=pltpu.MemorySpace.SMEM),
        scratch_shapes=[
            pltpu.VMEM((N * NP, P), jnp.float32),
            pltpu.VMEM((N * NP, P), jnp.float32),
            pltpu.VMEM((NP, P), jnp.float32),
            pltpu.VMEM((NP, P), jnp.float32),
            pltpu.SemaphoreType.DMA,
        ],
    )(*args)
    return out[0, 0]
